# initial kernel scaffold (unmeasured)
import functools

import jax
import jax.numpy as jnp
from jax import lax
from jax.experimental import pallas as pl
from jax.experimental.pallas import tpu as pltpu

N_DEV = 32

_sem_signal = getattr(pl, "semaphore_signal", None) or pltpu.semaphore_signal
_sem_wait = getattr(pl, "semaphore_wait", None) or pltpu.semaphore_wait
_CompilerParams = getattr(pltpu, "CompilerParams", None) or pltpu.TPUCompilerParams


def kernel(x, w_mat, scale_x, scale_w):
    m_total, k_per = x.shape
    k_total, n = w_mat.shape
    m_per = m_total // N_DEV
    comm_dtype = jnp.float8_e5m2

    def body(x_ref, w_ref, sx_ref, sw_ref, out_ref,
             x8_ref, xg_ref, w16_ref, send_sems, recv_sems):
        my = lax.axis_index("i")

        barrier = pltpu.get_barrier_semaphore()
        for d in range(1, N_DEV):
            _sem_signal(barrier, inc=1, device_id=((my + d) % N_DEV,),
                        device_id_type=pl.DeviceIdType.MESH)
        _sem_wait(barrier, N_DEV - 1)

        x8_ref[:, :] = x_ref[:, :].astype(comm_dtype)

        xg_ref[:, pl.ds(my * k_per, k_per)] = x8_ref[pl.ds(my * m_per, m_per), :]

        sends = []
        for d in range(1, N_DEV):
            dst = (my + d) % N_DEV
            rdma = pltpu.make_async_remote_copy(
                src_ref=x8_ref.at[pl.ds(dst * m_per, m_per), :],
                dst_ref=xg_ref.at[:, pl.ds(my * k_per, k_per)],
                send_sem=send_sems.at[d - 1],
                recv_sem=recv_sems.at[d - 1],
                device_id=(dst,),
                device_id_type=pl.DeviceIdType.MESH,
            )
            rdma.start()
            sends.append(rdma)

        w16_ref[:, :] = w_ref[:, :].astype(jnp.bfloat16)

        for d in range(1, N_DEV):
            src = (my - d) % N_DEV
            recv = pltpu.make_async_remote_copy(
                src_ref=x8_ref.at[pl.ds(src * m_per, m_per), :],
                dst_ref=xg_ref.at[:, pl.ds(src * k_per, k_per)],
                send_sem=send_sems.at[d - 1],
                recv_sem=recv_sems.at[d - 1],
                device_id=(src,),
                device_id_type=pl.DeviceIdType.MESH,
            )
            recv.wait_recv()
        for rdma in sends:
            rdma.wait_send()

        s = sx_ref[0] * sw_ref[0]
        acc = jnp.dot(xg_ref[:, :].astype(jnp.bfloat16), w16_ref[:, :],
                      preferred_element_type=jnp.float32)
        out_ref[:, :] = jnp.maximum(acc * s, 0.0)

        @functools.partial(pl.run_scoped, exit_sem=pltpu.SemaphoreType.REGULAR)
        def _(exit_sem):
            for d in range(1, N_DEV):
                _sem_signal(exit_sem, inc=1, device_id=((my + d) % N_DEV,),
                            device_id_type=pl.DeviceIdType.MESH)
            _sem_wait(exit_sem, N_DEV - 1)

    return pl.pallas_call(
        body,
        out_shape=jax.ShapeDtypeStruct((m_per, n), jnp.float32),
        in_specs=[
            pl.BlockSpec(memory_space=pltpu.VMEM),
            pl.BlockSpec(memory_space=pltpu.VMEM),
            pl.BlockSpec(memory_space=pltpu.SMEM),
            pl.BlockSpec(memory_space=pltpu.SMEM),
        ],
        out_specs=pl.BlockSpec(memory_space=pltpu.VMEM),
        scratch_shapes=[
            pltpu.VMEM((m_total, k_per), comm_dtype),
            pltpu.VMEM((m_per, k_total), comm_dtype),
            pltpu.VMEM((k_total, n), jnp.bfloat16),
            pltpu.SemaphoreType.DMA((N_DEV - 1,)),
            pltpu.SemaphoreType.DMA((N_DEV - 1,)),
        ],
        compiler_params=_CompilerParams(collective_id=0),
    )(x, w_mat, scale_x, scale_w)


# baseline (device time: 32576 ns/iter reference)
import functools

import jax
import jax.numpy as jnp
from jax import lax
from jax.experimental import pallas as pl
from jax.experimental.pallas import tpu as pltpu

N_DEV = 32
K_CHUNK = 512

_sem_signal = getattr(pl, "semaphore_signal", None) or pltpu.semaphore_signal
_sem_wait = getattr(pl, "semaphore_wait", None) or pltpu.semaphore_wait
_CompilerParams = getattr(pltpu, "CompilerParams", None) or pltpu.TPUCompilerParams


def kernel(x, w_mat, scale_x, scale_w):
    m_total, k_per = x.shape
    k_total, n = w_mat.shape
    m_per = m_total // N_DEV
    comm_dtype = jnp.float8_e5m2
    n_chunks = k_total // K_CHUNK
    blocks_per_chunk = K_CHUNK // k_per

    def body(x_ref, w_hbm, sx_ref, sw_ref, out_ref,
             x8_ref, xg_ref, wbuf_ref, wc16_ref,
             send_sems, recv_sems, w_sems):
        my = lax.axis_index("i")

        w_dmas = {}
        for c in range(min(2, n_chunks)):
            dma = pltpu.make_async_copy(
                w_hbm.at[pl.ds(c * K_CHUNK, K_CHUNK), :],
                wbuf_ref.at[c % 2],
                w_sems.at[c % 2],
            )
            dma.start()
            w_dmas[c] = dma

        x8_ref[:, :] = x_ref[:, :].astype(comm_dtype)

        loc = pltpu.make_async_copy(
            x8_ref.at[pl.ds(my * m_per, m_per), :],
            xg_ref.at[:, pl.ds(my * k_per, k_per)],
            recv_sems.at[my],
        )
        loc.start()

        barrier = pltpu.get_barrier_semaphore()
        for d in range(1, N_DEV):
            _sem_signal(barrier, inc=1, device_id=((my + d) % N_DEV,),
                        device_id_type=pl.DeviceIdType.MESH)
        _sem_wait(barrier, N_DEV - 1)

        sends = []
        for d in range(1, N_DEV):
            dst = (my + d) % N_DEV
            rdma = pltpu.make_async_remote_copy(
                src_ref=x8_ref.at[pl.ds(dst * m_per, m_per), :],
                dst_ref=xg_ref.at[:, pl.ds(my * k_per, k_per)],
                send_sem=send_sems.at[d - 1],
                recv_sem=recv_sems.at[my],
                device_id=(dst,),
                device_id_type=pl.DeviceIdType.MESH,
            )
            rdma.start()
            sends.append(rdma)

        s = sx_ref[0] * sw_ref[0]

        for c in range(n_chunks):
            w_dmas[c].wait()
            nxt = c + 2
            if nxt < n_chunks:
                dma = pltpu.make_async_copy(
                    w_hbm.at[pl.ds(nxt * K_CHUNK, K_CHUNK), :],
                    wbuf_ref.at[nxt % 2],
                    w_sems.at[nxt % 2],
                )
                dma.start()
                w_dmas[nxt] = dma
            wc16_ref[c % 2] = wbuf_ref[c % 2].astype(jnp.bfloat16)

            for j in range(c * blocks_per_chunk, (c + 1) * blocks_per_chunk):
                pltpu.make_async_copy(
                    x8_ref.at[pl.ds(0, m_per), :],
                    xg_ref.at[:, pl.ds(j * k_per, k_per)],
                    recv_sems.at[j],
                ).wait()

            part = jnp.dot(
                xg_ref[:, pl.ds(c * K_CHUNK, K_CHUNK)].astype(jnp.bfloat16),
                wc16_ref[c % 2],
                preferred_element_type=jnp.float32,
            )
            if c == 0:
                out_ref[:, :] = part
            else:
                out_ref[:, :] = out_ref[:, :] + part

        out_ref[:, :] = jnp.maximum(out_ref[:, :] * s, 0.0)

        for rdma in sends:
            rdma.wait_send()

        @functools.partial(pl.run_scoped, exit_sem=pltpu.SemaphoreType.REGULAR)
        def _(exit_sem):
            for d in range(1, N_DEV):
                _sem_signal(exit_sem, inc=1, device_id=((my + d) % N_DEV,),
                            device_id_type=pl.DeviceIdType.MESH)
            _sem_wait(exit_sem, N_DEV - 1)

    return pl.pallas_call(
        body,
        out_shape=jax.ShapeDtypeStruct((m_per, n), jnp.float32),
        in_specs=[
            pl.BlockSpec(memory_space=pltpu.VMEM),
            pl.BlockSpec(memory_space=pl.ANY),
            pl.BlockSpec(memory_space=pltpu.SMEM),
            pl.BlockSpec(memory_space=pltpu.SMEM),
        ],
        out_specs=pl.BlockSpec(memory_space=pltpu.VMEM),
        scratch_shapes=[
            pltpu.VMEM((m_total, k_per), comm_dtype),
            pltpu.VMEM((m_per, k_total), comm_dtype),
            pltpu.VMEM((2, K_CHUNK, n), jnp.float32),
            pltpu.VMEM((2, K_CHUNK, n), jnp.bfloat16),
            pltpu.SemaphoreType.DMA((N_DEV - 1,)),
            pltpu.SemaphoreType.DMA((N_DEV,)),
            pltpu.SemaphoreType.DMA((2,)),
        ],
        compiler_params=_CompilerParams(collective_id=0),
    )(x, w_mat, scale_x, scale_w)


# device time: 14494 ns/iter; 2.2476x vs baseline; 2.2476x over previous
import functools
import os

import jax
import jax.numpy as jnp
from jax import lax
from jax.experimental import pallas as pl
from jax.experimental.pallas import tpu as pltpu

N_DEV = 32
K_CHUNK = 512

_VARIANT = os.environ.get("KVARIANT", "full")

_sem_signal = getattr(pl, "semaphore_signal", None) or pltpu.semaphore_signal
_sem_wait = getattr(pl, "semaphore_wait", None) or pltpu.semaphore_wait
_CompilerParams = getattr(pltpu, "CompilerParams", None) or pltpu.TPUCompilerParams


def kernel(x, w_mat, scale_x, scale_w):
    m_total, k_per = x.shape
    k_total, n = w_mat.shape
    m_per = m_total // N_DEV
    comm_dtype = jnp.float8_e5m2
    n_chunks = k_total // K_CHUNK
    blocks_per_chunk = K_CHUNK // k_per

    def body(x_ref, w_hbm, sx_ref, sw_ref, out_ref,
             x8_ref, xg_ref, wbuf_ref, wc16_ref,
             send_sems, recv_sems, w_sems):
        my = lax.axis_index("i")

        w_dmas = {}
        for c in range(min(2, n_chunks)):
            dma = pltpu.make_async_copy(
                w_hbm.at[pl.ds(c * K_CHUNK, K_CHUNK), :],
                wbuf_ref.at[c % 2],
                w_sems.at[c % 2],
            )
            dma.start()
            w_dmas[c] = dma

        x8_ref[:, :] = x_ref[:, :].astype(comm_dtype)

        if _VARIANT == "local":
            xg_ref[:, pl.ds(my * k_per, k_per)] = x8_ref[pl.ds(my * m_per, m_per), :]
        else:
            loc = pltpu.make_async_copy(
                x8_ref.at[pl.ds(my * m_per, m_per), :],
                xg_ref.at[:, pl.ds(my * k_per, k_per)],
                recv_sems.at[my],
            )
            loc.start()

        sends = []
        if _VARIANT != "local":
            barrier = pltpu.get_barrier_semaphore()
            for d in range(1, N_DEV):
                _sem_signal(barrier, inc=1, device_id=((my + d) % N_DEV,),
                            device_id_type=pl.DeviceIdType.MESH)
            _sem_wait(barrier, N_DEV - 1)

            for d in range(1, N_DEV):
                dst = (my + d) % N_DEV
                rdma = pltpu.make_async_remote_copy(
                    src_ref=x8_ref.at[pl.ds(dst * m_per, m_per), :],
                    dst_ref=xg_ref.at[:, pl.ds(my * k_per, k_per)],
                    send_sem=send_sems.at[d - 1],
                    recv_sem=recv_sems.at[my],
                    device_id=(dst,),
                    device_id_type=pl.DeviceIdType.MESH,
                )
                rdma.start()
                sends.append(rdma)

        s = sx_ref[0] * sw_ref[0]

        for c in range(n_chunks):
            w_dmas[c].wait()
            nxt = c + 2
            if nxt < n_chunks:
                dma = pltpu.make_async_copy(
                    w_hbm.at[pl.ds(nxt * K_CHUNK, K_CHUNK), :],
                    wbuf_ref.at[nxt % 2],
                    w_sems.at[nxt % 2],
                )
                dma.start()
                w_dmas[nxt] = dma
            wc16_ref[c % 2] = wbuf_ref[c % 2].astype(jnp.bfloat16)

            if _VARIANT != "local":
                for j in range(c * blocks_per_chunk, (c + 1) * blocks_per_chunk):
                    pltpu.make_async_copy(
                        x8_ref.at[pl.ds(0, m_per), :],
                        xg_ref.at[:, pl.ds(j * k_per, k_per)],
                        recv_sems.at[j],
                    ).wait()

            part = jnp.dot(
                xg_ref[:, pl.ds(c * K_CHUNK, K_CHUNK)].astype(jnp.bfloat16),
                wc16_ref[c % 2],
                preferred_element_type=jnp.float32,
            )
            if c == 0:
                out_ref[:, :] = part
            else:
                out_ref[:, :] = out_ref[:, :] + part

        out_ref[:, :] = jnp.maximum(out_ref[:, :] * s, 0.0)

        for rdma in sends:
            rdma.wait_send()

        if _VARIANT == "full":
            @functools.partial(pl.run_scoped, exit_sem=pltpu.SemaphoreType.REGULAR)
            def _(exit_sem):
                for d in range(1, N_DEV):
                    _sem_signal(exit_sem, inc=1, device_id=((my + d) % N_DEV,),
                                device_id_type=pl.DeviceIdType.MESH)
                _sem_wait(exit_sem, N_DEV - 1)

    return pl.pallas_call(
        body,
        out_shape=jax.ShapeDtypeStruct((m_per, n), jnp.float32),
        in_specs=[
            pl.BlockSpec(memory_space=pltpu.VMEM),
            pl.BlockSpec(memory_space=pl.ANY),
            pl.BlockSpec(memory_space=pltpu.SMEM),
            pl.BlockSpec(memory_space=pltpu.SMEM),
        ],
        out_specs=pl.BlockSpec(memory_space=pltpu.VMEM),
        scratch_shapes=[
            pltpu.VMEM((m_total, k_per), comm_dtype),
            pltpu.VMEM((m_per, k_total), comm_dtype),
            pltpu.VMEM((2, K_CHUNK, n), jnp.float32),
            pltpu.VMEM((2, K_CHUNK, n), jnp.bfloat16),
            pltpu.SemaphoreType.DMA((N_DEV - 1,)),
            pltpu.SemaphoreType.DMA((N_DEV,)),
            pltpu.SemaphoreType.DMA((2,)),
        ],
        compiler_params=(_CompilerParams() if _VARIANT == "local"
                         else _CompilerParams(collective_id=0)),
    )(x, w_mat, scale_x, scale_w)
